# Initial kernel scaffold; baseline (speedup 1.0000x reference)
import jax
import jax.numpy as jnp
from jax.experimental import pallas as pl


def kernel(y, context_ptr, graph_h, W, b):
    raise NotImplementedError("write your pallas kernel here")


# speedup vs baseline: 1.3440x; 1.3440x over previous
import jax
import jax.numpy as jnp
from jax.experimental import pallas as pl
from jax.experimental.pallas import tpu as pltpu

_N = 131072
_C = 128
_B = 16
_ROWS = 1024


def _inject_body(idx_ref, y_ref, g_ref, wt_ref, b_ref, out_ref):
    i = pl.program_id(0)
    rows = jax.lax.broadcasted_iota(jnp.int32, (_ROWS, 1), 0) + i * _ROWS
    mask = jnp.ones((_ROWS, 1), jnp.float32)
    for k in range(_B):
        mask = jnp.where(rows == idx_ref[k], 0.0, mask)
    enc = (y_ref[...] * wt_ref[...] + b_ref[...]) * mask
    out_ref[:, :_C] = g_ref[:, :_C]
    out_ref[:, _C:] = g_ref[:, _C:] + enc


def kernel(y, context_ptr, graph_h, W, b):
    idx = context_ptr[1:].astype(jnp.int32) - 1
    idx = jnp.where(idx < 0, idx + _N, idx)
    g2 = graph_h.reshape(_N, 2 * _C)
    wt = W.reshape(1, _C)
    b2 = b.reshape(1, _C)

    grid_spec = pltpu.PrefetchScalarGridSpec(
        num_scalar_prefetch=1,
        grid=(_N // _ROWS,),
        in_specs=[
            pl.BlockSpec((_ROWS, 1), lambda i, idx_ref: (i, 0)),
            pl.BlockSpec((_ROWS, 2 * _C), lambda i, idx_ref: (i, 0)),
            pl.BlockSpec((1, _C), lambda i, idx_ref: (0, 0)),
            pl.BlockSpec((1, _C), lambda i, idx_ref: (0, 0)),
        ],
        out_specs=pl.BlockSpec((_ROWS, 2 * _C), lambda i, idx_ref: (i, 0)),
    )
    out = pl.pallas_call(
        _inject_body,
        grid_spec=grid_spec,
        out_shape=jax.ShapeDtypeStruct((_N, 2 * _C), jnp.float32),
    )(idx, y, g2, wt, b2)
    return out.reshape(2 * _N, _C)


# speedup vs baseline: 1.5379x; 1.1442x over previous
import jax
import jax.numpy as jnp
from jax.experimental import pallas as pl
from jax.experimental.pallas import tpu as pltpu

_N = 131072
_C = 128
_B = 16
_ROWS = 2048


def _inject_body(idx_ref, y_ref, g_ref, wt_ref, b_ref, out_ref):
    i = pl.program_id(0)
    lo = i * _ROWS
    out_ref[:, :_C] = g_ref[:, :_C]
    out_ref[:, _C:] = g_ref[:, _C:] + (y_ref[...] * wt_ref[...] + b_ref[...])
    for k in range(_B):
        r = idx_ref[k] - lo

        @pl.when((r >= 0) & (r < _ROWS))
        def _():
            w = (jnp.clip(r, 0, _ROWS - 1) // 8) * 8
            rows8 = jax.lax.broadcasted_iota(jnp.int32, (8, 1), 0) + (lo + w)
            m = jnp.ones((8, 1), jnp.float32)
            for j in range(_B):
                m = jnp.where(rows8 == idx_ref[j], 0.0, m)
            yw = y_ref[pl.ds(w, 8), :]
            out_ref[pl.ds(w, 8), _C:] = g_ref[pl.ds(w, 8), _C:] + m * (
                yw * wt_ref[...] + b_ref[...]
            )


def kernel(y, context_ptr, graph_h, W, b):
    idx = context_ptr[1:].astype(jnp.int32) - 1
    idx = jnp.where(idx < 0, idx + _N, idx)
    g2 = graph_h.reshape(_N, 2 * _C)
    wt = W.reshape(1, _C)
    b2 = b.reshape(1, _C)

    grid_spec = pltpu.PrefetchScalarGridSpec(
        num_scalar_prefetch=1,
        grid=(_N // _ROWS,),
        in_specs=[
            pl.BlockSpec((_ROWS, 1), lambda i, idx_ref: (i, 0)),
            pl.BlockSpec((_ROWS, 2 * _C), lambda i, idx_ref: (i, 0)),
            pl.BlockSpec((1, _C), lambda i, idx_ref: (0, 0)),
            pl.BlockSpec((1, _C), lambda i, idx_ref: (0, 0)),
        ],
        out_specs=pl.BlockSpec((_ROWS, 2 * _C), lambda i, idx_ref: (i, 0)),
    )
    out = pl.pallas_call(
        _inject_body,
        grid_spec=grid_spec,
        out_shape=jax.ShapeDtypeStruct((_N, 2 * _C), jnp.float32),
    )(idx, y, g2, wt, b2)
    return out.reshape(2 * _N, _C)


# speedup vs baseline: 2.2648x; 1.4727x over previous
import jax
import jax.numpy as jnp
from jax.experimental import pallas as pl
from jax.experimental.pallas import tpu as pltpu

_N = 131072
_C = 128
_B = 16
_BR = 4096


def _inject_body(oidx_ref, y2_ref, g_ref, wt_ref, b_ref, out_ref):
    i = pl.program_id(0)
    lo = i * _BR
    rows = jax.lax.broadcasted_iota(jnp.int32, (_BR, 1), 0)
    parity = (rows & 1).astype(jnp.float32)
    enc = (y2_ref[...] * wt_ref[...] + b_ref[...]) * parity
    out_ref[...] = g_ref[...] + enc
    for k in range(_B):
        r = oidx_ref[k] - lo

        @pl.when((r >= 0) & (r < _BR))
        def _():
            w = (jnp.clip(r, 0, _BR - 1) // 8) * 8
            rows8 = jax.lax.broadcasted_iota(jnp.int32, (8, 1), 0) + (lo + w)
            m = (rows8 & 1).astype(jnp.float32)
            for j in range(_B):
                m = jnp.where(rows8 == oidx_ref[j], 0.0, m)
            yw = y2_ref[pl.ds(w, 8), :]
            out_ref[pl.ds(w, 8), :] = g_ref[pl.ds(w, 8), :] + m * (
                yw * wt_ref[...] + b_ref[...]
            )


def kernel(y, context_ptr, graph_h, W, b):
    idx = context_ptr[1:].astype(jnp.int32) - 1
    idx = jnp.where(idx < 0, idx + _N, idx)
    oidx = 2 * idx + 1
    y2 = jnp.repeat(y, 2, axis=0)
    wt = W.reshape(1, _C)
    b2 = b.reshape(1, _C)

    grid_spec = pltpu.PrefetchScalarGridSpec(
        num_scalar_prefetch=1,
        grid=(2 * _N // _BR,),
        in_specs=[
            pl.BlockSpec((_BR, 1), lambda i, oidx_ref: (i, 0)),
            pl.BlockSpec((_BR, _C), lambda i, oidx_ref: (i, 0)),
            pl.BlockSpec((1, _C), lambda i, oidx_ref: (0, 0)),
            pl.BlockSpec((1, _C), lambda i, oidx_ref: (0, 0)),
        ],
        out_specs=pl.BlockSpec((_BR, _C), lambda i, oidx_ref: (i, 0)),
    )
    out = pl.pallas_call(
        _inject_body,
        grid_spec=grid_spec,
        out_shape=jax.ShapeDtypeStruct((2 * _N, _C), jnp.float32),
    )(oidx, y2, graph_h, wt, b2)
    return out
